# baseline (speedup 1.0000x reference)
import math

import jax
import jax.numpy as jnp
from jax.experimental import pallas as pl
from jax.experimental.pallas import tpu as pltpu

_N = 1024
_NFEAT = 128
_HIDDEN = 64
_NCLASS = 40
_NUM_LAYERS = 2
_ALPHA = 0.1
_THETA = 0.5
_C = 8
_BLK = _N // _C


def _mm(a, b):
    return jax.lax.dot_general(a, b, (((1,), (0,)), ((), ())),
                               precision=jax.lax.Precision.DEFAULT)


def _split_cat(hf):
    hb = hf.astype(jnp.bfloat16)
    hr = (hf - hb.astype(jnp.float32)).astype(jnp.bfloat16)
    return jnp.concatenate([hb, hr], axis=1)


def _magg(maskb, hcat):
    return jax.lax.dot_general(maskb, hcat, (((0,), (0,)), ((), ())),
                               preferred_element_type=jnp.float32)


def _gcn2_fwd(x_ref, adj_ref, w0_ref, b0_ref, w1_ref, b1_ref, cw_ref,
              out_ref, adj_vmem, maskb_ref, sem):
    def chunk_copy(c):
        return pltpu.make_async_copy(
            adj_ref.at[pl.ds(c * _BLK, _BLK), :],
            adj_vmem.at[pl.ds(c * _BLK, _BLK), :],
            sem.at[c])

    for c in range(_C):
        chunk_copy(c).start()

    h = jnp.maximum(_mm(x_ref[...], w0_ref[...]) + b0_ref[...], 0.0)
    x0 = h
    hcat = _split_cat(h)

    z = jnp.float32(_N * _N)
    agg = jnp.zeros((_N, 2 * _HIDDEN), jnp.float32)
    for c in range(_C):
        chunk_copy(c).wait()
        rows = pl.ds(c * _BLK, _BLK)
        nz = adj_vmem[rows, :] != 0.0
        maskb = nz.astype(jnp.bfloat16)
        maskb_ref[rows, :] = maskb
        z = z - jnp.sum(nz.astype(jnp.float32))
        agg = agg + _magg(maskb, hcat[c * _BLK:(c + 1) * _BLK, :])

    row_is0 = jax.lax.broadcasted_iota(jnp.int32, (_N, 1), 0) == 0

    def layer_update(o, h_prev, layer):
        beta = math.log(_THETA / (layer + 1) + 1.0)
        agg2 = o[:, :_HIDDEN] + o[:, _HIDDEN:]
        agg2 = agg2 + jnp.where(row_is0, z * h_prev[0:1, :], 0.0)
        out = agg2 * (1.0 - _ALPHA) + _ALPHA * x0
        out = (1.0 - beta) * out + beta * _mm(out, cw_ref[layer])
        return jnp.maximum(out, 0.0)

    h1 = layer_update(agg, h, 0)
    h2 = layer_update(_magg(maskb_ref[...], _split_cat(h1)), h1, 1)

    logits = _mm(h2, w1_ref[...]) + b1_ref[...]
    m = jnp.max(logits, axis=-1, keepdims=True)
    s = logits - m
    lse = jnp.log(jnp.sum(jnp.exp(s), axis=-1, keepdims=True))
    out_ref[...] = s - lse


def kernel(x, adj_t, lin0_w, lin0_b, lin1_w, lin1_b, conv_w):
    b0 = lin0_b.reshape(1, _HIDDEN)
    b1 = lin1_b.reshape(1, _NCLASS)
    vmem = pl.BlockSpec(memory_space=pltpu.VMEM)
    return pl.pallas_call(
        _gcn2_fwd,
        in_specs=[
            vmem,
            pl.BlockSpec(memory_space=pl.ANY),
            vmem, vmem, vmem, vmem, vmem,
        ],
        out_specs=vmem,
        out_shape=jax.ShapeDtypeStruct((_N, _NCLASS), jnp.float32),
        scratch_shapes=[
            pltpu.VMEM((_N, _N), jnp.float32),
            pltpu.VMEM((_N, _N), jnp.bfloat16),
            pltpu.SemaphoreType.DMA((_C,)),
        ],
    )(x, adj_t, lin0_w, b0, lin1_w, b1, conv_w)


# speedup vs baseline: 1.1103x; 1.1103x over previous
import math

import jax
import jax.numpy as jnp
from jax.experimental import pallas as pl

_N = 1024
_NFEAT = 128
_HIDDEN = 64
_NCLASS = 40
_NUM_LAYERS = 2
_ALPHA = 0.1
_THETA = 0.5


def _gcn2_fwd(x_ref, adj_ref, w0_ref, b0_ref, w1_ref, b1_ref, cw_ref, out_ref):
    def mm(a, b, dims):
        return jax.lax.dot_general(a, b, (dims, ((), ())),
                                   precision=jax.lax.Precision.DEFAULT)

    x = x_ref[...]
    h = jnp.maximum(mm(x, w0_ref[...], ((1,), (0,))) + b0_ref[...], 0.0)
    x0 = h

    adj = adj_ref[...]
    mask = (adj != 0.0).astype(jnp.float32)
    z = jnp.float32(_N * _N) - jnp.sum(mask)
    row_is0 = jax.lax.broadcasted_iota(jnp.int32, (_N, 1), 0) == 0

    def masked_agg(hf):
        return jax.lax.dot_general(mask, hf, ((((0,), (0,))), ((), ())),
                                   precision=jax.lax.Precision.DEFAULT)

    for layer in range(_NUM_LAYERS):
        beta = math.log(_THETA / (layer + 1) + 1.0)
        agg = masked_agg(h)
        agg = agg + jnp.where(row_is0, z * h[0:1, :], 0.0)
        out = agg * (1.0 - _ALPHA) + _ALPHA * x0
        out = (1.0 - beta) * out + beta * mm(out, cw_ref[layer], ((1,), (0,)))
        h = jnp.maximum(out, 0.0)

    logits = mm(h, w1_ref[...], ((1,), (0,))) + b1_ref[...]
    m = jnp.max(logits, axis=-1, keepdims=True)
    s = logits - m
    lse = jnp.log(jnp.sum(jnp.exp(s), axis=-1, keepdims=True))
    out_ref[...] = s - lse


def kernel(x, adj_t, lin0_w, lin0_b, lin1_w, lin1_b, conv_w):
    b0 = lin0_b.reshape(1, _HIDDEN)
    b1 = lin1_b.reshape(1, _NCLASS)
    return pl.pallas_call(
        _gcn2_fwd,
        out_shape=jax.ShapeDtypeStruct((_N, _NCLASS), jnp.float32),
    )(x, adj_t, lin0_w, b0, lin1_w, b1, conv_w)
